# Initial kernel scaffold; baseline (speedup 1.0000x reference)
import functools

import jax
import jax.numpy as jnp
from jax import lax
from jax.experimental import pallas as pl
from jax.experimental.pallas import tpu as pltpu
from jax.experimental.pallas import tpu_sc as plsc

N = 10000
E = 320000
D = 128
NC = 2
NS = 16
NW = NC * NS
EPW = E // NW
C = 80
CH = EPW // C
RPT = N // NS
DENR = 640
DRPT = DENR // NS
EPS = 1e-16


def _make_sc_kernel():
    mesh = plsc.VectorSubcoreMesh(core_axis_name="c", subcore_axis_name="s",
                                  num_cores=NC, num_subcores=NS)

    @functools.partial(
        pl.kernel,
        out_type=[
            jax.ShapeDtypeStruct((NC, N, D), jnp.float32),
            jax.ShapeDtypeStruct((NC, DENR, 16), jnp.float32),
        ],
        mesh=mesh,
        scratch_types=[
            pltpu.VMEM((N,), jnp.float32),
            pltpu.VMEM((N,), jnp.float32),
            pltpu.VMEM((EPW,), jnp.int32),
            pltpu.VMEM((EPW,), jnp.int32),
            pltpu.VMEM((CH, C), jnp.int32),
            pltpu.VMEM((C,), jnp.float32),
            pltpu.VMEM((C, 16), jnp.float32),
            pltpu.VMEM((C,), jnp.int32),
            pltpu.VMEM((1, C), jnp.int32),
            pltpu.VMEM((C, D), jnp.float32),
            pltpu.VMEM_SHARED((N, D), jnp.float32),
            pltpu.VMEM_SHARED((DENR, 16), jnp.float32),
        ],
    )
    def sc_kernel(z_hbm, sd_hbm, src1_hbm, dst1_hbm, dst2_hbm, zrows_hbm,
                  zden_hbm, raw_hbm, den_hbm,
                  s_loc, d_loc, src_loc, dst_loc, dst2_loc, ex_buf, exm_loc,
                  col_buf, dmrow_buf, rows_loc, raw_sh, den_sh):
        c = lax.axis_index("c")
        s = lax.axis_index("s")
        wid = c * NS + s

        pltpu.sync_copy(sd_hbm.at[0], s_loc)
        pltpu.sync_copy(sd_hbm.at[1], d_loc)
        pltpu.sync_copy(src1_hbm.at[wid], src_loc)
        pltpu.sync_copy(dst1_hbm.at[wid], dst_loc)
        pltpu.sync_copy(dst2_hbm.at[pl.ds(wid * CH, CH)], dst2_loc)

        pltpu.sync_copy(zrows_hbm, raw_sh.at[pl.ds(s * RPT, RPT)])
        pltpu.sync_copy(zden_hbm, den_sh.at[pl.ds(s * DRPT, DRPT)])
        zv = jnp.zeros((16,), jnp.float32)
        iota16 = lax.iota(jnp.int32, 16)

        def zero_body(i, _):
            exm_loc[i, pl.ds(0, 16)] = zv
            return 0

        lax.fori_loop(0, C, zero_body, 0)
        plsc.subcore_barrier()

        def chunk_body(ch, _):
            base = ch * C
            for j in range(C // 16):
                off = base + j * 16
                sidx = plsc.load_gather(src_loc, [off + iota16])
                didx = plsc.load_gather(dst_loc, [off + iota16])
                sv = plsc.load_gather(s_loc, [sidx])
                dv = plsc.load_gather(d_loc, [didx])
                t = sv + dv
                ex = jnp.exp(jnp.maximum(t, 0.2 * t))
                ex_buf[pl.ds(j * 16, 16)] = ex
                col = didx & 15
                plsc.store_scatter(exm_loc, [j * 16 + iota16, col], ex)
                col_buf[pl.ds(j * 16, 16)] = col
                dmrow_buf[0, pl.ds(j * 16, 16)] = didx >> 4
            pltpu.sync_copy(exm_loc, den_sh.at[dmrow_buf.at[0]], add=True)
            for j in range(C // 16):
                cidx = col_buf[pl.ds(j * 16, 16)]
                plsc.store_scatter(exm_loc, [j * 16 + iota16, cidx], zv)
            pltpu.sync_copy(z_hbm.at[src_loc.at[pl.ds(base, C)]], rows_loc)

            def scale_body(e, _):
                a = ex_buf[e]
                for f in range(D // 16):
                    rows_loc[e, pl.ds(f * 16, 16)] = (
                        rows_loc[e, pl.ds(f * 16, 16)] * a)
                return 0

            lax.fori_loop(0, C, scale_body, 0)
            pltpu.sync_copy(rows_loc, raw_sh.at[dst2_loc.at[ch]], add=True)
            return 0

        lax.fori_loop(0, CH, chunk_body, 0)

        plsc.subcore_barrier()
        pltpu.sync_copy(raw_sh.at[pl.ds(s * RPT, RPT)],
                        raw_hbm.at[c, pl.ds(s * RPT, RPT)])
        pltpu.sync_copy(den_sh.at[pl.ds(s * DRPT, DRPT)],
                        den_hbm.at[c, pl.ds(s * DRPT, DRPT)])

    return sc_kernel


_sc_layer = _make_sc_kernel()


def _first_body(x_ref, w_ref, a_ref, z_ref, sd_ref):
    z = jnp.dot(x_ref[...], w_ref[...], preferred_element_type=jnp.float32)
    z_ref[...] = z
    sd_ref[...] = lax.dot_general(a_ref[...], z, (((1,), (1,)), ((), ())),
                                  preferred_element_type=jnp.float32)


def _combine_body(raw_ref, den_ref, w_ref, a_ref, z_ref, sd_ref):
    dsum = den_ref[0, :] + den_ref[1, :]
    h = (raw_ref[0] + raw_ref[1]) / (dsum + EPS)[:, None]
    h = jnp.maximum(h, 0.0)
    z = jnp.dot(h, w_ref[...], preferred_element_type=jnp.float32)
    z_ref[...] = z
    sd_ref[...] = lax.dot_general(a_ref[...], z, (((1,), (1,)), ((), ())),
                                  preferred_element_type=jnp.float32)


def _final_body(raw_ref, den_ref, out_ref):
    dsum = den_ref[0, :] + den_ref[1, :]
    out_ref[...] = (raw_ref[0] + raw_ref[1]) / (dsum + EPS)[:, None]


_RB = 2000
_GRID = N // _RB


def _tc_first(x, W, A2):
    return pl.pallas_call(
        _first_body,
        grid=(_GRID,),
        in_specs=[
            pl.BlockSpec((_RB, D), lambda i: (i, 0)),
            pl.BlockSpec((D, D), lambda i: (0, 0)),
            pl.BlockSpec((2, D), lambda i: (0, 0)),
        ],
        out_specs=[
            pl.BlockSpec((_RB, D), lambda i: (i, 0)),
            pl.BlockSpec((2, _RB), lambda i: (0, i)),
        ],
        out_shape=[
            jax.ShapeDtypeStruct((N, D), jnp.float32),
            jax.ShapeDtypeStruct((2, N), jnp.float32),
        ],
    )(x, W, A2)


def _tc_combine(raw, den, W, A2):
    return pl.pallas_call(
        _combine_body,
        grid=(_GRID,),
        in_specs=[
            pl.BlockSpec((2, _RB, D), lambda i: (0, i, 0)),
            pl.BlockSpec((2, _RB), lambda i: (0, i)),
            pl.BlockSpec((D, D), lambda i: (0, 0)),
            pl.BlockSpec((2, D), lambda i: (0, 0)),
        ],
        out_specs=[
            pl.BlockSpec((_RB, D), lambda i: (i, 0)),
            pl.BlockSpec((2, _RB), lambda i: (0, i)),
        ],
        out_shape=[
            jax.ShapeDtypeStruct((N, D), jnp.float32),
            jax.ShapeDtypeStruct((2, N), jnp.float32),
        ],
    )(raw, den, W, A2)


def _tc_final(raw, den):
    return pl.pallas_call(
        _final_body,
        grid=(_GRID,),
        in_specs=[
            pl.BlockSpec((2, _RB, D), lambda i: (0, i, 0)),
            pl.BlockSpec((2, _RB), lambda i: (0, i)),
        ],
        out_specs=pl.BlockSpec((_RB, D), lambda i: (i, 0)),
        out_shape=jax.ShapeDtypeStruct((N, D), jnp.float32),
    )(raw, den)


def kernel(x, edge_index, W1, a1_src, a1_dst, W2, a2_src, a2_dst,
           W3, a3_src, a3_dst, W4, a4_src, a4_dst):
    src1 = edge_index[0].reshape(NW, EPW)
    dst1 = edge_index[1].reshape(NW, EPW)
    dst2 = edge_index[1].reshape(NW * CH, C)
    zrows = jnp.zeros((RPT, D), jnp.float32)
    zden = jnp.zeros((DRPT, 16), jnp.float32)

    As = [jnp.stack([a1_src, a1_dst]), jnp.stack([a2_src, a2_dst]),
          jnp.stack([a3_src, a3_dst]), jnp.stack([a4_src, a4_dst])]
    Ws = [W1, W2, W3, W4]

    z, sd = _tc_first(x, Ws[0], As[0])
    for layer in range(4):
        raw, den = _sc_layer(z, sd, src1, dst1, dst2, zrows, zden)
        den = den.reshape(NC, DENR * 16)
        if layer < 3:
            z, sd = _tc_combine(raw, den, Ws[layer + 1], As[layer + 1])
        else:
            return _tc_final(raw, den)


# speedup vs baseline: 17.1418x; 17.1418x over previous
import functools

import jax
import jax.numpy as jnp
from jax import lax
from jax.experimental import pallas as pl
from jax.experimental.pallas import tpu as pltpu
from jax.experimental.pallas import tpu_sc as plsc

N = 10000
E = 320000
D = 128
NC = 2
NS = 16
NW = NC * NS
EPW = E // NW
C = 80
CH = EPW // C
RPT = N // NS
DENR = 640
DRPT = DENR // NS
EPS = 1e-16


def _make_sc_kernel():
    mesh = plsc.VectorSubcoreMesh(core_axis_name="c", subcore_axis_name="s",
                                  num_cores=NC, num_subcores=NS)

    @functools.partial(
        pl.kernel,
        out_type=[
            jax.ShapeDtypeStruct((NC, NS, RPT, D), jnp.float32),
            jax.ShapeDtypeStruct((NC, DENR, 16), jnp.float32),
        ],
        mesh=mesh,
        compiler_params=pltpu.CompilerParams(needs_layout_passes=False),
        scratch_types=[
            pltpu.VMEM((2, C), jnp.int32),
            pltpu.VMEM((C,), jnp.float32),
            pltpu.VMEM((C,), jnp.float32),
            pltpu.VMEM((C,), jnp.float32),
            pltpu.VMEM((C, 16), jnp.float32),
            pltpu.VMEM((C,), jnp.int32),
            pltpu.VMEM((1, C), jnp.int32),
            pltpu.VMEM((C, D), jnp.float32),
            pltpu.VMEM_SHARED((N, D), jnp.float32),
            pltpu.VMEM_SHARED((DENR, 16), jnp.float32),
        ],
    )
    def sc_kernel(z_hbm, s_hbm, d_hbm, ei_hbm, zrows_hbm,
                  zden_hbm, raw_hbm, den_hbm,
                  idx_buf, sg_buf, dg_buf, ex_buf, exm_loc,
                  col_buf, dmrow_buf, rows_loc, raw_sh, den_sh):
        c = lax.axis_index("c")
        s = lax.axis_index("s")
        wid = c * NS + s

        pltpu.sync_copy(zrows_hbm, raw_sh.at[pl.ds(s * RPT, RPT)])
        pltpu.sync_copy(zden_hbm, den_sh.at[pl.ds(s * DRPT, DRPT)])
        zv = jnp.zeros((16,), jnp.float32)
        iota16 = lax.iota(jnp.int32, 16)

        def zero_body(i, _):
            exm_loc[i, pl.ds(0, 16)] = zv
            return 0

        lax.fori_loop(0, C, zero_body, 0)
        plsc.subcore_barrier()

        def chunk_body(ch, _):
            pltpu.sync_copy(ei_hbm.at[wid, ch], idx_buf)
            pltpu.sync_copy(s_hbm.at[idx_buf.at[0]], sg_buf)
            pltpu.sync_copy(d_hbm.at[idx_buf.at[1]], dg_buf)
            for j in range(C // 16):
                didx = idx_buf[1, pl.ds(j * 16, 16)]
                sv = sg_buf[pl.ds(j * 16, 16)]
                dv = dg_buf[pl.ds(j * 16, 16)]
                t = sv + dv
                ex = jnp.exp(jnp.maximum(t, 0.2 * t))
                ex_buf[pl.ds(j * 16, 16)] = ex
                col = didx & 15
                plsc.store_scatter(exm_loc, [j * 16 + iota16, col], ex)
                col_buf[pl.ds(j * 16, 16)] = col
                dmrow_buf[0, pl.ds(j * 16, 16)] = didx >> 4
            pltpu.sync_copy(exm_loc, den_sh.at[dmrow_buf.at[0]], add=True)
            for j in range(C // 16):
                cidx = col_buf[pl.ds(j * 16, 16)]
                plsc.store_scatter(exm_loc, [j * 16 + iota16, cidx], zv)
            pltpu.sync_copy(z_hbm.at[idx_buf.at[0]], rows_loc)

            def scale_body(e, _):
                av = plsc.load_gather(ex_buf, [jnp.broadcast_to(e, (16,))])
                for f in range(D // 16):
                    rows_loc[e, pl.ds(f * 16, 16)] = (
                        rows_loc[e, pl.ds(f * 16, 16)] * av)
                return 0

            lax.fori_loop(0, C, scale_body, 0)
            pltpu.sync_copy(rows_loc, raw_sh.at[idx_buf.at[1]], add=True)
            return 0

        lax.fori_loop(0, CH, chunk_body, 0)

        plsc.subcore_barrier()
        pltpu.sync_copy(raw_sh.at[pl.ds(s * RPT, RPT)], raw_hbm.at[c, s])
        pltpu.sync_copy(den_sh.at[pl.ds(s * DRPT, DRPT)],
                        den_hbm.at[c, pl.ds(s * DRPT, DRPT)])

    return sc_kernel


_sc_layer = _make_sc_kernel()


def _first_body(x_ref, w_ref, a_ref, z_ref, sd_ref):
    z = jnp.dot(x_ref[...], w_ref[...], preferred_element_type=jnp.float32)
    z_ref[...] = z
    sd_ref[...] = lax.dot_general(a_ref[...], z, (((1,), (1,)), ((), ())),
                                  preferred_element_type=jnp.float32)


def _combine_body(raw_ref, den_ref, w_ref, a_ref, z_ref, sd_ref):
    dsum = den_ref[0, :] + den_ref[1, :]
    h = (raw_ref[0] + raw_ref[1]) / (dsum + EPS)[:, None]
    h = jnp.maximum(h, 0.0)
    z = jnp.dot(h, w_ref[...], preferred_element_type=jnp.float32)
    z_ref[...] = z
    sd_ref[...] = lax.dot_general(a_ref[...], z, (((1,), (1,)), ((), ())),
                                  preferred_element_type=jnp.float32)


def _final_body(raw_ref, den_ref, out_ref):
    dsum = den_ref[0, :] + den_ref[1, :]
    out_ref[...] = (raw_ref[0] + raw_ref[1]) / (dsum + EPS)[:, None]


_RB = 2048
_GRID = (N + _RB - 1) // _RB


def _tc_first(x, W, A2):
    return pl.pallas_call(
        _first_body,
        grid=(_GRID,),
        in_specs=[
            pl.BlockSpec((_RB, D), lambda i: (i, 0)),
            pl.BlockSpec((D, D), lambda i: (0, 0)),
            pl.BlockSpec((2, D), lambda i: (0, 0)),
        ],
        out_specs=[
            pl.BlockSpec((_RB, D), lambda i: (i, 0)),
            pl.BlockSpec((2, _RB), lambda i: (0, i)),
        ],
        out_shape=[
            jax.ShapeDtypeStruct((N, D), jnp.float32),
            jax.ShapeDtypeStruct((2, N), jnp.float32),
        ],
    )(x, W, A2)


def _tc_combine(raw, den, W, A2):
    return pl.pallas_call(
        _combine_body,
        grid=(_GRID,),
        in_specs=[
            pl.BlockSpec((2, _RB, D), lambda i: (0, i, 0)),
            pl.BlockSpec((2, _RB), lambda i: (0, i)),
            pl.BlockSpec((D, D), lambda i: (0, 0)),
            pl.BlockSpec((2, D), lambda i: (0, 0)),
        ],
        out_specs=[
            pl.BlockSpec((_RB, D), lambda i: (i, 0)),
            pl.BlockSpec((2, _RB), lambda i: (0, i)),
        ],
        out_shape=[
            jax.ShapeDtypeStruct((N, D), jnp.float32),
            jax.ShapeDtypeStruct((2, N), jnp.float32),
        ],
    )(raw, den, W, A2)


def _tc_final(raw, den):
    return pl.pallas_call(
        _final_body,
        grid=(_GRID,),
        in_specs=[
            pl.BlockSpec((2, _RB, D), lambda i: (0, i, 0)),
            pl.BlockSpec((2, _RB), lambda i: (0, i)),
        ],
        out_specs=pl.BlockSpec((_RB, D), lambda i: (i, 0)),
        out_shape=jax.ShapeDtypeStruct((N, D), jnp.float32),
    )(raw, den)


def kernel(x, edge_index, W1, a1_src, a1_dst, W2, a2_src, a2_dst,
           W3, a3_src, a3_dst, W4, a4_src, a4_dst):
    ei = jnp.transpose(edge_index.reshape(2, NW, CH, C), (1, 2, 0, 3))
    zrows = jnp.zeros((RPT, D), jnp.float32)
    zden = jnp.zeros((DRPT, 16), jnp.float32)

    As = [jnp.stack([a1_src, a1_dst]), jnp.stack([a2_src, a2_dst]),
          jnp.stack([a3_src, a3_dst]), jnp.stack([a4_src, a4_dst])]
    Ws = [W1, W2, W3, W4]

    z, sd = _tc_first(x, Ws[0], As[0])
    for layer in range(4):
        raw, den = _sc_layer(z, sd[0], sd[1], ei, zrows, zden)
        raw = raw.reshape(NC, N, D)
        den = den.reshape(NC, DENR * 16)
        if layer < 3:
            z, sd = _tc_combine(raw, den, Ws[layer + 1], As[layer + 1])
        else:
            return _tc_final(raw, den)


# speedup vs baseline: 35.5832x; 2.0758x over previous
import functools

import jax
import jax.numpy as jnp
from jax import lax
from jax.experimental import pallas as pl
from jax.experimental.pallas import tpu as pltpu
from jax.experimental.pallas import tpu_sc as plsc

N = 10000
E = 320000
D = 128
NC = 2
NS = 16
NW = NC * NS
EPW = E // NW
C = 80
CH = EPW // C
RPT = N // NS
DENR = 640
DRPT = DENR // NS
EPS = 1e-16


def _make_sc_kernel():
    mesh = plsc.VectorSubcoreMesh(core_axis_name="c", subcore_axis_name="s",
                                  num_cores=NC, num_subcores=NS)

    @functools.partial(
        pl.kernel,
        out_type=[
            jax.ShapeDtypeStruct((NC, NS, RPT, D), jnp.float32),
            jax.ShapeDtypeStruct((NC, DENR, 16), jnp.float32),
        ],
        mesh=mesh,
        compiler_params=pltpu.CompilerParams(needs_layout_passes=False),
        scratch_types=[
            pltpu.VMEM((2, C), jnp.int32),
            pltpu.VMEM((2, C), jnp.int32),
            pltpu.VMEM((C,), jnp.float32),
            pltpu.VMEM((C,), jnp.float32),
            pltpu.VMEM((C,), jnp.float32),
            pltpu.VMEM((C,), jnp.float32),
            pltpu.VMEM((C, D), jnp.float32),
            pltpu.VMEM((C, D), jnp.float32),
            pltpu.VMEM((C,), jnp.float32),
            pltpu.VMEM((C, 16), jnp.float32),
            pltpu.VMEM((C,), jnp.int32),
            pltpu.VMEM((1, C), jnp.int32),
            pltpu.VMEM_SHARED((N, D), jnp.float32),
            pltpu.VMEM_SHARED((DENR, 16), jnp.float32),
            pltpu.SemaphoreType.DMA,
            pltpu.SemaphoreType.DMA,
            pltpu.SemaphoreType.DMA,
            pltpu.SemaphoreType.DMA,
            pltpu.SemaphoreType.DMA,
            pltpu.SemaphoreType.DMA,
            pltpu.SemaphoreType.DMA,
            pltpu.SemaphoreType.DMA,
        ],
    )
    def sc_kernel(z_hbm, s_hbm, d_hbm, ei_hbm, zrows_hbm,
                  zden_hbm, raw_hbm, den_hbm,
                  idx_a, idx_b, sg_a, sg_b, dg_a, dg_b, rows_a, rows_b,
                  ex_buf, exm_loc, col_buf, dmrow_buf, raw_sh, den_sh,
                  gsem_a, gsem_b, ssem_a, ssem_b, dsem_a, dsem_b,
                  sctsem_a, sctsem_b):
        c = lax.axis_index("c")
        s = lax.axis_index("s")
        wid = c * NS + s
        buf_a = (idx_a, sg_a, dg_a, rows_a, gsem_a, ssem_a, dsem_a, sctsem_a)
        buf_b = (idx_b, sg_b, dg_b, rows_b, gsem_b, ssem_b, dsem_b, sctsem_b)

        pltpu.sync_copy(zrows_hbm, raw_sh.at[pl.ds(s * RPT, RPT)])
        pltpu.sync_copy(zden_hbm, den_sh.at[pl.ds(s * DRPT, DRPT)])
        zv = jnp.zeros((16,), jnp.float32)
        iota16 = lax.iota(jnp.int32, 16)

        def zero_body(i, _):
            exm_loc[i, pl.ds(0, 16)] = zv
            return 0

        lax.fori_loop(0, C, zero_body, 0)
        plsc.subcore_barrier()

        def issue_gathers(b):
            idx, sg, dg, rows, gsem, ssem, dsem, _ = b
            pltpu.async_copy(z_hbm.at[idx.at[0]], rows, gsem)
            pltpu.async_copy(s_hbm.at[idx.at[0]], sg, ssem)
            pltpu.async_copy(d_hbm.at[idx.at[1]], dg, dsem)

        def wait_sct(b):
            idx, _, _, rows, _, _, _, sctsem = b
            pltpu.make_async_copy(rows, raw_sh.at[idx.at[1]], sctsem).wait()

        def refill(b, ch):
            idx = b[0]
            pltpu.sync_copy(ei_hbm.at[wid, ch], idx)
            issue_gathers(b)

        def process(b, mid=None):
            idx, sg, dg, rows, gsem, ssem, dsem, sctsem = b
            pltpu.make_async_copy(s_hbm.at[idx.at[0]], sg, ssem).wait()
            pltpu.make_async_copy(d_hbm.at[idx.at[1]], dg, dsem).wait()
            for j in range(C // 16):
                didx = idx[1, pl.ds(j * 16, 16)]
                sv = sg[pl.ds(j * 16, 16)]
                dv = dg[pl.ds(j * 16, 16)]
                t = sv + dv
                ex = jnp.exp(jnp.maximum(t, 0.2 * t))
                ex_buf[pl.ds(j * 16, 16)] = ex
                col = didx & 15
                plsc.store_scatter(exm_loc, [j * 16 + iota16, col], ex)
                col_buf[pl.ds(j * 16, 16)] = col
                dmrow_buf[0, pl.ds(j * 16, 16)] = didx >> 4
            pltpu.sync_copy(exm_loc, den_sh.at[dmrow_buf.at[0]], add=True)
            for j in range(C // 16):
                cidx = col_buf[pl.ds(j * 16, 16)]
                plsc.store_scatter(exm_loc, [j * 16 + iota16, cidx], zv)
            if mid is not None:
                mid()
            pltpu.make_async_copy(z_hbm.at[idx.at[0]], rows, gsem).wait()

            def scale_body(e, _):
                av = plsc.load_gather(ex_buf, [jnp.broadcast_to(e, (16,))])
                for f in range(D // 16):
                    rows[e, pl.ds(f * 16, 16)] = (
                        rows[e, pl.ds(f * 16, 16)] * av)
                return 0

            lax.fori_loop(0, C, scale_body, 0)
            pltpu.async_copy(rows, raw_sh.at[idx.at[1]], sctsem, add=True)

        refill(buf_a, 0)

        def body(g, _):
            ch0 = 2 * g

            @pl.when(g > 0)
            def _():
                wait_sct(buf_b)

            refill(buf_b, ch0 + 1)
            process(buf_a)

            def mid():
                wait_sct(buf_a)
                refill(buf_a, ch0 + 2)

            process(buf_b, mid=mid)
            return 0

        lax.fori_loop(0, CH // 2, body, 0)
        wait_sct(buf_b)
        process(buf_a)
        wait_sct(buf_a)

        plsc.subcore_barrier()
        pltpu.sync_copy(raw_sh.at[pl.ds(s * RPT, RPT)], raw_hbm.at[c, s])
        pltpu.sync_copy(den_sh.at[pl.ds(s * DRPT, DRPT)],
                        den_hbm.at[c, pl.ds(s * DRPT, DRPT)])

    return sc_kernel


_sc_layer = _make_sc_kernel()


def _first_body(x_ref, w_ref, a_ref, z_ref, sd_ref):
    z = jnp.dot(x_ref[...], w_ref[...], preferred_element_type=jnp.float32)
    z_ref[...] = z
    sd_ref[...] = lax.dot_general(a_ref[...], z, (((1,), (1,)), ((), ())),
                                  preferred_element_type=jnp.float32)


def _combine_body(raw_ref, den_ref, w_ref, a_ref, z_ref, sd_ref):
    dsum = den_ref[0, :] + den_ref[1, :]
    h = (raw_ref[0] + raw_ref[1]) / (dsum + EPS)[:, None]
    h = jnp.maximum(h, 0.0)
    z = jnp.dot(h, w_ref[...], preferred_element_type=jnp.float32)
    z_ref[...] = z
    sd_ref[...] = lax.dot_general(a_ref[...], z, (((1,), (1,)), ((), ())),
                                  preferred_element_type=jnp.float32)


def _final_body(raw_ref, den_ref, out_ref):
    dsum = den_ref[0, :] + den_ref[1, :]
    out_ref[...] = (raw_ref[0] + raw_ref[1]) / (dsum + EPS)[:, None]


_RB = 2048
_GRID = (N + _RB - 1) // _RB


def _tc_first(x, W, A2):
    return pl.pallas_call(
        _first_body,
        grid=(_GRID,),
        in_specs=[
            pl.BlockSpec((_RB, D), lambda i: (i, 0)),
            pl.BlockSpec((D, D), lambda i: (0, 0)),
            pl.BlockSpec((2, D), lambda i: (0, 0)),
        ],
        out_specs=[
            pl.BlockSpec((_RB, D), lambda i: (i, 0)),
            pl.BlockSpec((2, _RB), lambda i: (0, i)),
        ],
        out_shape=[
            jax.ShapeDtypeStruct((N, D), jnp.float32),
            jax.ShapeDtypeStruct((2, N), jnp.float32),
        ],
    )(x, W, A2)


def _tc_combine(raw, den, W, A2):
    return pl.pallas_call(
        _combine_body,
        grid=(_GRID,),
        in_specs=[
            pl.BlockSpec((2, _RB, D), lambda i: (0, i, 0)),
            pl.BlockSpec((2, _RB), lambda i: (0, i)),
            pl.BlockSpec((D, D), lambda i: (0, 0)),
            pl.BlockSpec((2, D), lambda i: (0, 0)),
        ],
        out_specs=[
            pl.BlockSpec((_RB, D), lambda i: (i, 0)),
            pl.BlockSpec((2, _RB), lambda i: (0, i)),
        ],
        out_shape=[
            jax.ShapeDtypeStruct((N, D), jnp.float32),
            jax.ShapeDtypeStruct((2, N), jnp.float32),
        ],
    )(raw, den, W, A2)


def _tc_final(raw, den):
    return pl.pallas_call(
        _final_body,
        grid=(_GRID,),
        in_specs=[
            pl.BlockSpec((2, _RB, D), lambda i: (0, i, 0)),
            pl.BlockSpec((2, _RB), lambda i: (0, i)),
        ],
        out_specs=pl.BlockSpec((_RB, D), lambda i: (i, 0)),
        out_shape=jax.ShapeDtypeStruct((N, D), jnp.float32),
    )(raw, den)


def kernel(x, edge_index, W1, a1_src, a1_dst, W2, a2_src, a2_dst,
           W3, a3_src, a3_dst, W4, a4_src, a4_dst):
    ei = jnp.transpose(edge_index.reshape(2, NW, CH, C), (1, 2, 0, 3))
    zrows = jnp.zeros((RPT, D), jnp.float32)
    zden = jnp.zeros((DRPT, 16), jnp.float32)

    As = [jnp.stack([a1_src, a1_dst]), jnp.stack([a2_src, a2_dst]),
          jnp.stack([a3_src, a3_dst]), jnp.stack([a4_src, a4_dst])]
    Ws = [W1, W2, W3, W4]

    z, sd = _tc_first(x, Ws[0], As[0])
    for layer in range(4):
        raw, den = _sc_layer(z, sd[0], sd[1], ei, zrows, zden)
        raw = raw.reshape(NC, N, D)
        den = den.reshape(NC, DENR * 16)
        if layer < 3:
            z, sd = _tc_combine(raw, den, Ws[layer + 1], As[layer + 1])
        else:
            return _tc_final(raw, den)


# speedup vs baseline: 42.8586x; 1.2045x over previous
import functools

import jax
import jax.numpy as jnp
from jax import lax
from jax.experimental import pallas as pl
from jax.experimental.pallas import tpu as pltpu
from jax.experimental.pallas import tpu_sc as plsc

N = 10000
E = 320000
D = 128
NC = 2
NS = 16
NW = NC * NS
EPW = E // NW
C = 80
CH = EPW // C
RPT = N // NS
DENR = 640
DRPT = DENR // NS
EPS = 1e-16


def _make_sc_kernel():
    mesh = plsc.VectorSubcoreMesh(core_axis_name="c", subcore_axis_name="s",
                                  num_cores=NC, num_subcores=NS)

    @functools.partial(
        pl.kernel,
        out_type=[
            jax.ShapeDtypeStruct((NC, NS, RPT, D), jnp.float32),
            jax.ShapeDtypeStruct((NC, DENR, 16), jnp.float32),
        ],
        mesh=mesh,
        compiler_params=pltpu.CompilerParams(needs_layout_passes=False),
        scratch_types=[
            pltpu.VMEM((2, C), jnp.int32),
            pltpu.VMEM((2, C), jnp.int32),
            pltpu.VMEM((C,), jnp.float32),
            pltpu.VMEM((C,), jnp.float32),
            pltpu.VMEM((C,), jnp.float32),
            pltpu.VMEM((C,), jnp.float32),
            pltpu.VMEM((C, D), jnp.float32),
            pltpu.VMEM((C, D), jnp.float32),
            pltpu.VMEM((C,), jnp.float32),
            pltpu.VMEM((C, 16), jnp.float32),
            pltpu.VMEM((C,), jnp.int32),
            pltpu.VMEM((1, C), jnp.int32),
            pltpu.VMEM_SHARED((N, D), jnp.float32),
            pltpu.VMEM_SHARED((DENR, 16), jnp.float32),
            pltpu.SemaphoreType.DMA,
            pltpu.SemaphoreType.DMA,
            pltpu.SemaphoreType.DMA,
            pltpu.SemaphoreType.DMA,
            pltpu.SemaphoreType.DMA,
            pltpu.SemaphoreType.DMA,
            pltpu.SemaphoreType.DMA,
            pltpu.SemaphoreType.DMA,
        ],
    )
    def sc_kernel(z_hbm, s_hbm, d_hbm, ei_hbm, zrows_hbm,
                  zden_hbm, raw_hbm, den_hbm,
                  idx_a, idx_b, sg_a, sg_b, dg_a, dg_b, rows_a, rows_b,
                  ex_buf, exm_loc, col_buf, dmrow_buf, raw_sh, den_sh,
                  gsem_a, gsem_b, ssem_a, ssem_b, dsem_a, dsem_b,
                  sctsem_a, sctsem_b):
        c = lax.axis_index("c")
        s = lax.axis_index("s")
        wid = c * NS + s
        buf_a = (idx_a, sg_a, dg_a, rows_a, gsem_a, ssem_a, dsem_a, sctsem_a)
        buf_b = (idx_b, sg_b, dg_b, rows_b, gsem_b, ssem_b, dsem_b, sctsem_b)

        pltpu.sync_copy(zrows_hbm, raw_sh.at[pl.ds(s * RPT, RPT)])
        pltpu.sync_copy(zden_hbm, den_sh.at[pl.ds(s * DRPT, DRPT)])
        zv = jnp.zeros((16,), jnp.float32)
        iota16 = lax.iota(jnp.int32, 16)

        def zero_body(i, _):
            exm_loc[i, pl.ds(0, 16)] = zv
            return 0

        lax.fori_loop(0, C, zero_body, 0)
        plsc.subcore_barrier()

        def issue_gathers(b):
            idx, sg, dg, rows, gsem, ssem, dsem, _ = b
            pltpu.async_copy(z_hbm.at[idx.at[0]], rows, gsem)
            pltpu.async_copy(s_hbm.at[idx.at[0]], sg, ssem)
            pltpu.async_copy(d_hbm.at[idx.at[1]], dg, dsem)

        def wait_sct(b):
            idx, _, _, rows, _, _, _, sctsem = b
            pltpu.make_async_copy(rows, raw_sh.at[idx.at[1]], sctsem).wait()

        def refill(b, ch):
            idx = b[0]
            pltpu.sync_copy(ei_hbm.at[wid, ch], idx)
            issue_gathers(b)

        def process(b, mid=None):
            idx, sg, dg, rows, gsem, ssem, dsem, sctsem = b
            pltpu.make_async_copy(s_hbm.at[idx.at[0]], sg, ssem).wait()
            pltpu.make_async_copy(d_hbm.at[idx.at[1]], dg, dsem).wait()
            for j in range(C // 16):
                didx = idx[1, pl.ds(j * 16, 16)]
                sv = sg[pl.ds(j * 16, 16)]
                dv = dg[pl.ds(j * 16, 16)]
                t = sv + dv
                ex = jnp.exp(jnp.maximum(t, 0.2 * t))
                ex_buf[pl.ds(j * 16, 16)] = ex
                col = didx & 15
                plsc.store_scatter(exm_loc, [j * 16 + iota16, col], ex)
                col_buf[pl.ds(j * 16, 16)] = col
                dmrow_buf[0, pl.ds(j * 16, 16)] = didx >> 4
            pltpu.sync_copy(exm_loc, den_sh.at[dmrow_buf.at[0]], add=True)
            for j in range(C // 16):
                cidx = col_buf[pl.ds(j * 16, 16)]
                plsc.store_scatter(exm_loc, [j * 16 + iota16, cidx], zv)
            if mid is not None:
                mid()
            pltpu.make_async_copy(z_hbm.at[idx.at[0]], rows, gsem).wait()

            def scale_body(g, _):
                exv = ex_buf[pl.ds(g * 16, 16)]
                for i in range(16):
                    av = jnp.take_along_axis(
                        exv, jnp.full((16,), i, jnp.int32), axis=0)
                    e = g * 16 + i
                    for f in range(D // 16):
                        rows[e, pl.ds(f * 16, 16)] = (
                            rows[e, pl.ds(f * 16, 16)] * av)
                return 0

            lax.fori_loop(0, C // 16, scale_body, 0)
            pltpu.async_copy(rows, raw_sh.at[idx.at[1]], sctsem, add=True)

        refill(buf_a, 0)

        def body(g, _):
            ch0 = 2 * g

            @pl.when(g > 0)
            def _():
                wait_sct(buf_b)

            refill(buf_b, ch0 + 1)
            process(buf_a)

            def mid():
                wait_sct(buf_a)
                refill(buf_a, ch0 + 2)

            process(buf_b, mid=mid)
            return 0

        lax.fori_loop(0, CH // 2, body, 0)
        wait_sct(buf_b)
        process(buf_a)
        wait_sct(buf_a)

        plsc.subcore_barrier()
        pltpu.sync_copy(raw_sh.at[pl.ds(s * RPT, RPT)], raw_hbm.at[c, s])
        pltpu.sync_copy(den_sh.at[pl.ds(s * DRPT, DRPT)],
                        den_hbm.at[c, pl.ds(s * DRPT, DRPT)])

    return sc_kernel


_sc_layer = _make_sc_kernel()


def _first_body(x_ref, w_ref, a_ref, z_ref, sd_ref):
    z = jnp.dot(x_ref[...], w_ref[...], preferred_element_type=jnp.float32)
    z_ref[...] = z
    sd_ref[...] = lax.dot_general(a_ref[...], z, (((1,), (1,)), ((), ())),
                                  preferred_element_type=jnp.float32)


def _combine_body(raw_ref, den_ref, w_ref, a_ref, z_ref, sd_ref):
    dsum = den_ref[0, :] + den_ref[1, :]
    h = (raw_ref[0] + raw_ref[1]) / (dsum + EPS)[:, None]
    h = jnp.maximum(h, 0.0)
    z = jnp.dot(h, w_ref[...], preferred_element_type=jnp.float32)
    z_ref[...] = z
    sd_ref[...] = lax.dot_general(a_ref[...], z, (((1,), (1,)), ((), ())),
                                  preferred_element_type=jnp.float32)


def _final_body(raw_ref, den_ref, out_ref):
    dsum = den_ref[0, :] + den_ref[1, :]
    out_ref[...] = (raw_ref[0] + raw_ref[1]) / (dsum + EPS)[:, None]


_RB = 2048
_GRID = (N + _RB - 1) // _RB


def _tc_first(x, W, A2):
    return pl.pallas_call(
        _first_body,
        grid=(_GRID,),
        in_specs=[
            pl.BlockSpec((_RB, D), lambda i: (i, 0)),
            pl.BlockSpec((D, D), lambda i: (0, 0)),
            pl.BlockSpec((2, D), lambda i: (0, 0)),
        ],
        out_specs=[
            pl.BlockSpec((_RB, D), lambda i: (i, 0)),
            pl.BlockSpec((2, _RB), lambda i: (0, i)),
        ],
        out_shape=[
            jax.ShapeDtypeStruct((N, D), jnp.float32),
            jax.ShapeDtypeStruct((2, N), jnp.float32),
        ],
    )(x, W, A2)


def _tc_combine(raw, den, W, A2):
    return pl.pallas_call(
        _combine_body,
        grid=(_GRID,),
        in_specs=[
            pl.BlockSpec((2, _RB, D), lambda i: (0, i, 0)),
            pl.BlockSpec((2, _RB), lambda i: (0, i)),
            pl.BlockSpec((D, D), lambda i: (0, 0)),
            pl.BlockSpec((2, D), lambda i: (0, 0)),
        ],
        out_specs=[
            pl.BlockSpec((_RB, D), lambda i: (i, 0)),
            pl.BlockSpec((2, _RB), lambda i: (0, i)),
        ],
        out_shape=[
            jax.ShapeDtypeStruct((N, D), jnp.float32),
            jax.ShapeDtypeStruct((2, N), jnp.float32),
        ],
    )(raw, den, W, A2)


def _tc_final(raw, den):
    return pl.pallas_call(
        _final_body,
        grid=(_GRID,),
        in_specs=[
            pl.BlockSpec((2, _RB, D), lambda i: (0, i, 0)),
            pl.BlockSpec((2, _RB), lambda i: (0, i)),
        ],
        out_specs=pl.BlockSpec((_RB, D), lambda i: (i, 0)),
        out_shape=jax.ShapeDtypeStruct((N, D), jnp.float32),
    )(raw, den)


def kernel(x, edge_index, W1, a1_src, a1_dst, W2, a2_src, a2_dst,
           W3, a3_src, a3_dst, W4, a4_src, a4_dst):
    ei = jnp.transpose(edge_index.reshape(2, NW, CH, C), (1, 2, 0, 3))
    zrows = jnp.zeros((RPT, D), jnp.float32)
    zden = jnp.zeros((DRPT, 16), jnp.float32)

    As = [jnp.stack([a1_src, a1_dst]), jnp.stack([a2_src, a2_dst]),
          jnp.stack([a3_src, a3_dst]), jnp.stack([a4_src, a4_dst])]
    Ws = [W1, W2, W3, W4]

    z, sd = _tc_first(x, Ws[0], As[0])
    for layer in range(4):
        raw, den = _sc_layer(z, sd[0], sd[1], ei, zrows, zden)
        raw = raw.reshape(NC, N, D)
        den = den.reshape(NC, DENR * 16)
        if layer < 3:
            z, sd = _tc_combine(raw, den, Ws[layer + 1], As[layer + 1])
        else:
            return _tc_final(raw, den)


# speedup vs baseline: 44.4163x; 1.0363x over previous
import functools

import jax
import jax.numpy as jnp
from jax import lax
from jax.experimental import pallas as pl
from jax.experimental.pallas import tpu as pltpu
from jax.experimental.pallas import tpu_sc as plsc

N = 10000
E = 320000
D = 128
NC = 2
NS = 16
NW = NC * NS
EPW = E // NW
C = 80
CH = EPW // C
RPT = N // NS
DENR = 640
DRPT = DENR // NS
EPS = 1e-16


def _make_sc_kernel():
    mesh = plsc.VectorSubcoreMesh(core_axis_name="c", subcore_axis_name="s",
                                  num_cores=NC, num_subcores=NS)

    @functools.partial(
        pl.kernel,
        out_type=[
            jax.ShapeDtypeStruct((NC, NS, RPT, D), jnp.float32),
            jax.ShapeDtypeStruct((NC, DENR, 16), jnp.float32),
        ],
        mesh=mesh,
        compiler_params=pltpu.CompilerParams(needs_layout_passes=False),
        scratch_types=[
            pltpu.VMEM((2, C), jnp.int32),
            pltpu.VMEM((2, C), jnp.int32),
            pltpu.VMEM((C,), jnp.float32),
            pltpu.VMEM((C,), jnp.float32),
            pltpu.VMEM((C,), jnp.float32),
            pltpu.VMEM((C,), jnp.float32),
            pltpu.VMEM((C, D), jnp.float32),
            pltpu.VMEM((C, D), jnp.float32),
            pltpu.VMEM((C,), jnp.float32),
            pltpu.VMEM((C, 16), jnp.float32),
            pltpu.VMEM((C, 16), jnp.float32),
            pltpu.VMEM((C,), jnp.int32),
            pltpu.VMEM((C,), jnp.int32),
            pltpu.VMEM((1, C), jnp.int32),
            pltpu.VMEM((1, C), jnp.int32),
            pltpu.VMEM_SHARED((N, D), jnp.float32),
            pltpu.VMEM_SHARED((DENR, 16), jnp.float32),
            pltpu.SemaphoreType.DMA,
            pltpu.SemaphoreType.DMA,
            pltpu.SemaphoreType.DMA,
            pltpu.SemaphoreType.DMA,
            pltpu.SemaphoreType.DMA,
            pltpu.SemaphoreType.DMA,
            pltpu.SemaphoreType.DMA,
            pltpu.SemaphoreType.DMA,
            pltpu.SemaphoreType.DMA,
            pltpu.SemaphoreType.DMA,
        ],
    )
    def sc_kernel(z_hbm, s_hbm, d_hbm, ei_hbm, zrows_hbm,
                  zden_hbm, raw_hbm, den_hbm,
                  idx_a, idx_b, sg_a, sg_b, dg_a, dg_b, rows_a, rows_b,
                  ex_buf, exm_a, exm_b, col_a, col_b, dmrow_a, dmrow_b,
                  raw_sh, den_sh,
                  gsem_a, gsem_b, ssem_a, ssem_b, dsem_a, dsem_b,
                  sctsem_a, sctsem_b, densem_a, densem_b):
        c = lax.axis_index("c")
        s = lax.axis_index("s")
        wid = c * NS + s
        buf_a = (idx_a, sg_a, dg_a, rows_a, gsem_a, ssem_a, dsem_a, sctsem_a,
                 exm_a, col_a, dmrow_a, densem_a)
        buf_b = (idx_b, sg_b, dg_b, rows_b, gsem_b, ssem_b, dsem_b, sctsem_b,
                 exm_b, col_b, dmrow_b, densem_b)

        pltpu.sync_copy(zrows_hbm, raw_sh.at[pl.ds(s * RPT, RPT)])
        pltpu.sync_copy(zden_hbm, den_sh.at[pl.ds(s * DRPT, DRPT)])
        zv = jnp.zeros((16,), jnp.float32)
        iota16 = lax.iota(jnp.int32, 16)

        def zero_body(i, _):
            exm_a[i, pl.ds(0, 16)] = zv
            exm_b[i, pl.ds(0, 16)] = zv
            return 0

        lax.fori_loop(0, C, zero_body, 0)
        plsc.subcore_barrier()

        def issue_gathers(b):
            idx, sg, dg, rows = b[:4]
            gsem, ssem, dsem = b[4:7]
            pltpu.async_copy(z_hbm.at[idx.at[0]], rows, gsem)
            pltpu.async_copy(s_hbm.at[idx.at[0]], sg, ssem)
            pltpu.async_copy(d_hbm.at[idx.at[1]], dg, dsem)

        def wait_sct(b):
            idx, rows, sctsem = b[0], b[3], b[7]
            pltpu.make_async_copy(rows, raw_sh.at[idx.at[1]], sctsem).wait()

        def wait_den(b):
            exm, dmrow, densem = b[8], b[10], b[11]
            pltpu.make_async_copy(exm, den_sh.at[dmrow.at[0]], densem).wait()

        def refill(b, ch):
            idx = b[0]
            pltpu.sync_copy(ei_hbm.at[wid, ch], idx)
            issue_gathers(b)

        def process(b, notfirst, mid=None):
            (idx, sg, dg, rows, gsem, ssem, dsem, sctsem,
             exm, colb, dmrow, densem) = b
            pltpu.make_async_copy(s_hbm.at[idx.at[0]], sg, ssem).wait()
            pltpu.make_async_copy(d_hbm.at[idx.at[1]], dg, dsem).wait()

            @pl.when(notfirst)
            def _():
                wait_den(b)
                for j in range(C // 16):
                    cidx = colb[pl.ds(j * 16, 16)]
                    plsc.store_scatter(exm, [j * 16 + iota16, cidx], zv)

            for j in range(C // 16):
                didx = idx[1, pl.ds(j * 16, 16)]
                sv = sg[pl.ds(j * 16, 16)]
                dv = dg[pl.ds(j * 16, 16)]
                t = sv + dv
                ex = jnp.exp(jnp.maximum(t, 0.2 * t))
                ex_buf[pl.ds(j * 16, 16)] = ex
                col = didx & 15
                plsc.store_scatter(exm, [j * 16 + iota16, col], ex)
                colb[pl.ds(j * 16, 16)] = col
                dmrow[0, pl.ds(j * 16, 16)] = didx >> 4
            pltpu.async_copy(exm, den_sh.at[dmrow.at[0]], densem, add=True)
            if mid is not None:
                mid()
            pltpu.make_async_copy(z_hbm.at[idx.at[0]], rows, gsem).wait()

            def scale_body(g, _):
                exv = ex_buf[pl.ds(g * 16, 16)]
                for i in range(16):
                    av = jnp.take_along_axis(
                        exv, jnp.full((16,), i, jnp.int32), axis=0)
                    e = g * 16 + i
                    for f in range(D // 16):
                        rows[e, pl.ds(f * 16, 16)] = (
                            rows[e, pl.ds(f * 16, 16)] * av)
                return 0

            lax.fori_loop(0, C // 16, scale_body, 0)
            pltpu.async_copy(rows, raw_sh.at[idx.at[1]], sctsem, add=True)

        refill(buf_a, 0)

        def body(g, _):
            ch0 = 2 * g

            @pl.when(g > 0)
            def _():
                wait_sct(buf_b)

            refill(buf_b, ch0 + 1)
            process(buf_a, g > 0)

            def mid():
                wait_sct(buf_a)
                refill(buf_a, ch0 + 2)

            process(buf_b, g > 0, mid=mid)
            return 0

        lax.fori_loop(0, CH // 2, body, 0)
        wait_sct(buf_b)
        process(buf_a, c >= 0)
        wait_sct(buf_a)
        wait_den(buf_a)
        wait_den(buf_b)

        plsc.subcore_barrier()
        pltpu.sync_copy(raw_sh.at[pl.ds(s * RPT, RPT)], raw_hbm.at[c, s])
        pltpu.sync_copy(den_sh.at[pl.ds(s * DRPT, DRPT)],
                        den_hbm.at[c, pl.ds(s * DRPT, DRPT)])

    return sc_kernel


_sc_layer = _make_sc_kernel()


def _first_body(x_ref, w_ref, a_ref, z_ref, sd_ref):
    z = jnp.dot(x_ref[...], w_ref[...], preferred_element_type=jnp.float32)
    z_ref[...] = z
    sd_ref[...] = lax.dot_general(a_ref[...], z, (((1,), (1,)), ((), ())),
                                  preferred_element_type=jnp.float32)


def _combine_body(raw_ref, den_ref, w_ref, a_ref, z_ref, sd_ref):
    dsum = den_ref[0, :] + den_ref[1, :]
    h = (raw_ref[0] + raw_ref[1]) / (dsum + EPS)[:, None]
    h = jnp.maximum(h, 0.0)
    z = jnp.dot(h, w_ref[...], preferred_element_type=jnp.float32)
    z_ref[...] = z
    sd_ref[...] = lax.dot_general(a_ref[...], z, (((1,), (1,)), ((), ())),
                                  preferred_element_type=jnp.float32)


def _final_body(raw_ref, den_ref, out_ref):
    dsum = den_ref[0, :] + den_ref[1, :]
    out_ref[...] = (raw_ref[0] + raw_ref[1]) / (dsum + EPS)[:, None]


_RB = 2048
_GRID = (N + _RB - 1) // _RB


def _tc_first(x, W, A2):
    return pl.pallas_call(
        _first_body,
        grid=(_GRID,),
        in_specs=[
            pl.BlockSpec((_RB, D), lambda i: (i, 0)),
            pl.BlockSpec((D, D), lambda i: (0, 0)),
            pl.BlockSpec((2, D), lambda i: (0, 0)),
        ],
        out_specs=[
            pl.BlockSpec((_RB, D), lambda i: (i, 0)),
            pl.BlockSpec((2, _RB), lambda i: (0, i)),
        ],
        out_shape=[
            jax.ShapeDtypeStruct((N, D), jnp.float32),
            jax.ShapeDtypeStruct((2, N), jnp.float32),
        ],
    )(x, W, A2)


def _tc_combine(raw, den, W, A2):
    return pl.pallas_call(
        _combine_body,
        grid=(_GRID,),
        in_specs=[
            pl.BlockSpec((2, _RB, D), lambda i: (0, i, 0)),
            pl.BlockSpec((2, _RB), lambda i: (0, i)),
            pl.BlockSpec((D, D), lambda i: (0, 0)),
            pl.BlockSpec((2, D), lambda i: (0, 0)),
        ],
        out_specs=[
            pl.BlockSpec((_RB, D), lambda i: (i, 0)),
            pl.BlockSpec((2, _RB), lambda i: (0, i)),
        ],
        out_shape=[
            jax.ShapeDtypeStruct((N, D), jnp.float32),
            jax.ShapeDtypeStruct((2, N), jnp.float32),
        ],
    )(raw, den, W, A2)


def _tc_final(raw, den):
    return pl.pallas_call(
        _final_body,
        grid=(_GRID,),
        in_specs=[
            pl.BlockSpec((2, _RB, D), lambda i: (0, i, 0)),
            pl.BlockSpec((2, _RB), lambda i: (0, i)),
        ],
        out_specs=pl.BlockSpec((_RB, D), lambda i: (i, 0)),
        out_shape=jax.ShapeDtypeStruct((N, D), jnp.float32),
    )(raw, den)


def kernel(x, edge_index, W1, a1_src, a1_dst, W2, a2_src, a2_dst,
           W3, a3_src, a3_dst, W4, a4_src, a4_dst):
    ei = jnp.transpose(edge_index.reshape(2, NW, CH, C), (1, 2, 0, 3))
    zrows = jnp.zeros((RPT, D), jnp.float32)
    zden = jnp.zeros((DRPT, 16), jnp.float32)

    As = [jnp.stack([a1_src, a1_dst]), jnp.stack([a2_src, a2_dst]),
          jnp.stack([a3_src, a3_dst]), jnp.stack([a4_src, a4_dst])]
    Ws = [W1, W2, W3, W4]

    z, sd = _tc_first(x, Ws[0], As[0])
    for layer in range(4):
        raw, den = _sc_layer(z, sd[0], sd[1], ei, zrows, zden)
        raw = raw.reshape(NC, N, D)
        den = den.reshape(NC, DENR * 16)
        if layer < 3:
            z, sd = _tc_combine(raw, den, Ws[layer + 1], As[layer + 1])
        else:
            return _tc_final(raw, den)


# speedup vs baseline: 46.0677x; 1.0372x over previous
import functools

import jax
import jax.numpy as jnp
from jax import lax
from jax.experimental import pallas as pl
from jax.experimental.pallas import tpu as pltpu
from jax.experimental.pallas import tpu_sc as plsc

N = 10000
E = 320000
D = 128
NC = 2
NS = 16
NW = NC * NS
EPW = E // NW
C = 80
CH = EPW // C
RPT = N // NS
DEN_PAD = 10240
DPT = DEN_PAD // NS
EPS = 1e-16


def _make_sc_kernel():
    mesh = plsc.VectorSubcoreMesh(core_axis_name="c", subcore_axis_name="s",
                                  num_cores=NC, num_subcores=NS)

    @functools.partial(
        pl.kernel,
        out_type=[
            jax.ShapeDtypeStruct((NC, NS, RPT, D), jnp.float32),
            jax.ShapeDtypeStruct((NC, DEN_PAD), jnp.float32),
        ],
        mesh=mesh,
        compiler_params=pltpu.CompilerParams(needs_layout_passes=False),
        scratch_types=[
            pltpu.VMEM((2, C), jnp.int32),
            pltpu.VMEM((2, C), jnp.int32),
            pltpu.VMEM((C,), jnp.float32),
            pltpu.VMEM((C,), jnp.float32),
            pltpu.VMEM((C,), jnp.float32),
            pltpu.VMEM((C,), jnp.float32),
            pltpu.VMEM((C, D), jnp.float32),
            pltpu.VMEM((C, D), jnp.float32),
            pltpu.VMEM((C,), jnp.float32),
            pltpu.VMEM((C,), jnp.float32),
            pltpu.VMEM_SHARED((N, D), jnp.float32),
            pltpu.VMEM_SHARED((DEN_PAD,), jnp.float32),
            pltpu.SemaphoreType.DMA,
            pltpu.SemaphoreType.DMA,
            pltpu.SemaphoreType.DMA,
            pltpu.SemaphoreType.DMA,
            pltpu.SemaphoreType.DMA,
            pltpu.SemaphoreType.DMA,
            pltpu.SemaphoreType.DMA,
            pltpu.SemaphoreType.DMA,
            pltpu.SemaphoreType.DMA,
            pltpu.SemaphoreType.DMA,
        ],
    )
    def sc_kernel(z_hbm, s_hbm, d_hbm, ei_hbm, zrows_hbm,
                  zden_hbm, raw_hbm, den_hbm,
                  idx_a, idx_b, sg_a, sg_b, dg_a, dg_b, rows_a, rows_b,
                  exv_a, exv_b, raw_sh, den_sh,
                  gsem_a, gsem_b, ssem_a, ssem_b, dsem_a, dsem_b,
                  sctsem_a, sctsem_b, densem_a, densem_b):
        c = lax.axis_index("c")
        s = lax.axis_index("s")
        wid = c * NS + s
        buf_a = (idx_a, sg_a, dg_a, rows_a, gsem_a, ssem_a, dsem_a, sctsem_a,
                 exv_a, densem_a)
        buf_b = (idx_b, sg_b, dg_b, rows_b, gsem_b, ssem_b, dsem_b, sctsem_b,
                 exv_b, densem_b)

        pltpu.sync_copy(zrows_hbm, raw_sh.at[pl.ds(s * RPT, RPT)])
        pltpu.sync_copy(zden_hbm, den_sh.at[pl.ds(s * DPT, DPT)])
        plsc.subcore_barrier()

        def issue_gathers(b):
            idx, sg, dg, rows = b[:4]
            gsem, ssem, dsem = b[4:7]
            pltpu.async_copy(z_hbm.at[idx.at[0]], rows, gsem)
            pltpu.async_copy(s_hbm.at[idx.at[0]], sg, ssem)
            pltpu.async_copy(d_hbm.at[idx.at[1]], dg, dsem)

        def wait_sct(b):
            idx, rows, sctsem = b[0], b[3], b[7]
            pltpu.make_async_copy(rows, raw_sh.at[idx.at[1]], sctsem).wait()

        def wait_den(b):
            idx, exv, densem = b[0], b[8], b[9]
            pltpu.make_async_copy(exv, den_sh.at[idx.at[1]], densem).wait()

        def refill(b, ch):
            idx = b[0]
            pltpu.sync_copy(ei_hbm.at[wid, ch], idx)
            issue_gathers(b)

        def process(b, mid=None):
            (idx, sg, dg, rows, gsem, ssem, dsem, sctsem, exv, densem) = b
            pltpu.make_async_copy(s_hbm.at[idx.at[0]], sg, ssem).wait()
            pltpu.make_async_copy(d_hbm.at[idx.at[1]], dg, dsem).wait()
            for j in range(C // 16):
                sv = sg[pl.ds(j * 16, 16)]
                dv = dg[pl.ds(j * 16, 16)]
                t = sv + dv
                exv[pl.ds(j * 16, 16)] = jnp.exp(jnp.maximum(t, 0.2 * t))
            pltpu.async_copy(exv, den_sh.at[idx.at[1]], densem, add=True)
            if mid is not None:
                mid()
            pltpu.make_async_copy(z_hbm.at[idx.at[0]], rows, gsem).wait()

            def scale_body(g, _):
                exg = exv[pl.ds(g * 16, 16)]
                for i in range(16):
                    av = jnp.take_along_axis(
                        exg, jnp.full((16,), i, jnp.int32), axis=0)
                    e = g * 16 + i
                    for f in range(D // 16):
                        rows[e, pl.ds(f * 16, 16)] = (
                            rows[e, pl.ds(f * 16, 16)] * av)
                return 0

            lax.fori_loop(0, C // 16, scale_body, 0)
            pltpu.async_copy(rows, raw_sh.at[idx.at[1]], sctsem, add=True)

        refill(buf_a, 0)

        def body(g, _):
            ch0 = 2 * g

            @pl.when(g > 0)
            def _():
                wait_sct(buf_b)
                wait_den(buf_b)

            refill(buf_b, ch0 + 1)
            process(buf_a)

            def mid():
                wait_sct(buf_a)
                wait_den(buf_a)
                refill(buf_a, ch0 + 2)

            process(buf_b, mid=mid)
            return 0

        lax.fori_loop(0, CH // 2, body, 0)
        wait_sct(buf_b)
        wait_den(buf_b)
        process(buf_a)
        wait_sct(buf_a)
        wait_den(buf_a)

        plsc.subcore_barrier()
        pltpu.sync_copy(raw_sh.at[pl.ds(s * RPT, RPT)], raw_hbm.at[c, s])
        pltpu.sync_copy(den_sh.at[pl.ds(s * DPT, DPT)],
                        den_hbm.at[c, pl.ds(s * DPT, DPT)])

    return sc_kernel


_sc_layer = _make_sc_kernel()


def _first_body(x_ref, w_ref, a_ref, z_ref, sd_ref):
    z = jnp.dot(x_ref[...], w_ref[...], preferred_element_type=jnp.float32)
    z_ref[...] = z
    sd_ref[...] = lax.dot_general(a_ref[...], z, (((1,), (1,)), ((), ())),
                                  preferred_element_type=jnp.float32)


def _combine_body(raw_ref, den_ref, w_ref, a_ref, z_ref, sd_ref):
    dsum = den_ref[0, :] + den_ref[1, :]
    h = (raw_ref[0] + raw_ref[1]) / (dsum + EPS)[:, None]
    h = jnp.maximum(h, 0.0)
    z = jnp.dot(h, w_ref[...], preferred_element_type=jnp.float32)
    z_ref[...] = z
    sd_ref[...] = lax.dot_general(a_ref[...], z, (((1,), (1,)), ((), ())),
                                  preferred_element_type=jnp.float32)


def _final_body(raw_ref, den_ref, out_ref):
    dsum = den_ref[0, :] + den_ref[1, :]
    out_ref[...] = (raw_ref[0] + raw_ref[1]) / (dsum + EPS)[:, None]


_RB = 2048
_GRID = (N + _RB - 1) // _RB


def _tc_first(x, W, A2):
    return pl.pallas_call(
        _first_body,
        grid=(_GRID,),
        in_specs=[
            pl.BlockSpec((_RB, D), lambda i: (i, 0)),
            pl.BlockSpec((D, D), lambda i: (0, 0)),
            pl.BlockSpec((2, D), lambda i: (0, 0)),
        ],
        out_specs=[
            pl.BlockSpec((_RB, D), lambda i: (i, 0)),
            pl.BlockSpec((2, _RB), lambda i: (0, i)),
        ],
        out_shape=[
            jax.ShapeDtypeStruct((N, D), jnp.float32),
            jax.ShapeDtypeStruct((2, N), jnp.float32),
        ],
    )(x, W, A2)


def _tc_combine(raw, den, W, A2):
    return pl.pallas_call(
        _combine_body,
        grid=(_GRID,),
        in_specs=[
            pl.BlockSpec((2, _RB, D), lambda i: (0, i, 0)),
            pl.BlockSpec((2, _RB), lambda i: (0, i)),
            pl.BlockSpec((D, D), lambda i: (0, 0)),
            pl.BlockSpec((2, D), lambda i: (0, 0)),
        ],
        out_specs=[
            pl.BlockSpec((_RB, D), lambda i: (i, 0)),
            pl.BlockSpec((2, _RB), lambda i: (0, i)),
        ],
        out_shape=[
            jax.ShapeDtypeStruct((N, D), jnp.float32),
            jax.ShapeDtypeStruct((2, N), jnp.float32),
        ],
    )(raw, den, W, A2)


def _tc_final(raw, den):
    return pl.pallas_call(
        _final_body,
        grid=(_GRID,),
        in_specs=[
            pl.BlockSpec((2, _RB, D), lambda i: (0, i, 0)),
            pl.BlockSpec((2, _RB), lambda i: (0, i)),
        ],
        out_specs=pl.BlockSpec((_RB, D), lambda i: (i, 0)),
        out_shape=jax.ShapeDtypeStruct((N, D), jnp.float32),
    )(raw, den)


def kernel(x, edge_index, W1, a1_src, a1_dst, W2, a2_src, a2_dst,
           W3, a3_src, a3_dst, W4, a4_src, a4_dst):
    ei = jnp.transpose(edge_index.reshape(2, NW, CH, C), (1, 2, 0, 3))
    zrows = jnp.zeros((RPT, D), jnp.float32)
    zden = jnp.zeros((DPT,), jnp.float32)

    As = [jnp.stack([a1_src, a1_dst]), jnp.stack([a2_src, a2_dst]),
          jnp.stack([a3_src, a3_dst]), jnp.stack([a4_src, a4_dst])]
    Ws = [W1, W2, W3, W4]

    z, sd = _tc_first(x, Ws[0], As[0])
    for layer in range(4):
        raw, den = _sc_layer(z, sd[0], sd[1], ei, zrows, zden)
        raw = raw.reshape(NC, N, D)
        if layer < 3:
            z, sd = _tc_combine(raw, den, Ws[layer + 1], As[layer + 1])
        else:
            return _tc_final(raw, den)
